# baseline (speedup 1.0000x reference)
import functools

import jax
import jax.numpy as jnp
from jax import lax
from jax.experimental import pallas as pl
from jax.experimental.pallas import tpu as pltpu
from jax.experimental.pallas import tpu_sc as plsc

L = 16
NW = 32
CHUNK = 2048


def _make_kernel(nq, nk):
    q_per_w = nq // NW
    n_chunks = q_per_w // CHUNK
    assert n_chunks % 2 == 0
    mesh = plsc.VectorSubcoreMesh(core_axis_name="c", subcore_axis_name="s")

    vm = lambda dt: pltpu.VMEM((CHUNK,), dt)

    @functools.partial(
        pl.kernel,
        out_type=jax.ShapeDtypeStruct((nq,), jnp.float32),
        mesh=mesh,
        scratch_types=[
            [vm(jnp.float32) for _ in range(2)],
            [vm(jnp.int32) for _ in range(2)],
            [vm(jnp.int32) for _ in range(2)],
            [vm(jnp.float32) for _ in range(2)],
            [vm(jnp.float32) for _ in range(2)],
            [pltpu.SemaphoreType.DMA for _ in range(2)],
            [vm(jnp.float32) for _ in range(2)],
            pltpu.VMEM_SHARED((nk,), jnp.float32),
        ],
    )
    def k(tq_hbm, y_hbm, out_hbm, t_v, i0_v, i1_v, v0_v, v1_v, gsem, o_v, y_sp):
        sid = lax.axis_index("s")

        @pl.when(sid == 0)
        def _():
            pltpu.sync_copy(y_hbm, y_sp)

        plsc.subcore_barrier()

        wid = lax.axis_index("s") * 2 + lax.axis_index("c")
        w_base = wid * q_per_w
        t_max = jnp.float32(nk - 1)
        i_max = jnp.int32(nk - 2)

        def stage_and_fire(c, b):
            base = w_base + c * CHUNK
            pltpu.sync_copy(tq_hbm.at[pl.ds(base, CHUNK)], t_v[b])

            for i in range(CHUNK // L):
                sl = pl.ds(i * L, L)
                t = t_v[b][sl]
                i0 = jnp.minimum(t.astype(jnp.int32), i_max)
                i0_v[b][sl] = i0
                i1_v[b][sl] = i0 + 1
                t_v[b][sl] = t - i0.astype(jnp.float32)
            pltpu.async_copy(y_sp.at[i0_v[b]], v0_v[b], gsem[b])
            pltpu.async_copy(y_sp.at[i1_v[b]], v1_v[b], gsem[b])

        def drain_and_store(c, b):
            pltpu.make_async_copy(y_sp.at[i0_v[b]], v0_v[b], gsem[b]).wait()
            pltpu.make_async_copy(y_sp.at[i1_v[b]], v1_v[b], gsem[b]).wait()

            for i in range(CHUNK // L):
                sl = pl.ds(i * L, L)
                v0 = v0_v[b][sl]
                v0_v[b][sl] = v0 + (v1_v[b][sl] - v0) * t_v[b][sl]

        stage_and_fire(0, 0)

        def pair_body(c2, _):
            s1 = 2 * c2 + 1

            @pl.when(s1 < n_chunks)
            def _():
                stage_and_fire(s1, 1)

            drain_and_store(s1 - 1, 0)

            @pl.when(s1 + 1 < n_chunks)
            def _():
                stage_and_fire(s1 + 1, 0)

            @pl.when(s1 < n_chunks)
            def _():
                drain_and_store(s1, 1)

            return 0

        lax.fori_loop(0, n_chunks // 2, pair_body, 0)

    return k


def kernel(t_query, t_knots, y):
    nq = t_query.shape[0]
    nk = t_knots.shape[0]
    return _make_kernel(nq, nk)(t_query, y)


# speedup vs baseline: 1.0476x; 1.0476x over previous
import functools

import jax
import jax.numpy as jnp
from jax import lax
from jax.experimental import pallas as pl
from jax.experimental.pallas import tpu as pltpu
from jax.experimental.pallas import tpu_sc as plsc

L = 16
NC = 2
NS = 16
CHUNK = 4096


def _make_kernel(nq, nk):
    piece = NS * CHUNK
    q_per_core = nq // NC
    n_chunks = q_per_core // piece
    assert n_chunks % 2 == 0 and n_chunks >= 4
    mesh = plsc.VectorSubcoreMesh(core_axis_name="c", subcore_axis_name="s")

    vm = lambda dt: pltpu.VMEM((CHUNK,), dt)

    @functools.partial(
        pl.kernel,
        out_type=jax.ShapeDtypeStruct((nq,), jnp.float32),
        mesh=mesh,
        scratch_types=[
            [vm(jnp.float32) for _ in range(2)],
            [vm(jnp.int32) for _ in range(2)],
            [vm(jnp.int32) for _ in range(2)],
            [vm(jnp.float32) for _ in range(2)],
            [vm(jnp.float32) for _ in range(2)],
            [pltpu.SemaphoreType.DMA for _ in range(2)],
            [pltpu.SemaphoreType.DMA for _ in range(2)],
            pltpu.VMEM_SHARED((nk,), jnp.float32),
            [pltpu.VMEM_SHARED((NS * CHUNK,), jnp.float32) for _ in range(2)],
        ],
    )
    def k(tq_hbm, y_hbm, out_hbm, t_v, i0_v, i1_v, v0_v, v1_v,
          gsem, tsem, y_sp, t_sp):
        sid = lax.axis_index("s")
        cid = lax.axis_index("c")
        core_base = cid * q_per_core
        i_max = jnp.int32(nk - 2)

        def bulk_src(c):
            return tq_hbm.at[pl.ds(core_base + c * piece, piece)]

        @pl.when(sid == 0)
        def _():
            pltpu.sync_copy(y_hbm, y_sp)
            pltpu.async_copy(bulk_src(0), t_sp[0], tsem[0])

        def stage_and_fire(c, b):
            @pl.when(sid == 0)
            def _():
                pltpu.make_async_copy(bulk_src(c), t_sp[b], tsem[b]).wait()

            plsc.subcore_barrier()

            @pl.when(jnp.logical_and(sid == 0, c + 1 < n_chunks))
            def _():
                pltpu.async_copy(bulk_src(c + 1), t_sp[1 - b], tsem[1 - b])

            pltpu.sync_copy(t_sp[b].at[pl.ds(sid * CHUNK, CHUNK)], t_v[b])

            def idx_body(i, _):
                sl = pl.ds(i * L, L)
                t = t_v[b][sl]
                i0 = jnp.minimum(t.astype(jnp.int32), i_max)
                i0_v[b][sl] = i0
                i1_v[b][sl] = i0 + 1
                t_v[b][sl] = t - i0.astype(jnp.float32)
                return 0

            lax.fori_loop(0, CHUNK // L, idx_body, 0, unroll=8)
            pltpu.async_copy(y_sp.at[i0_v[b]], v0_v[b], gsem[b])
            pltpu.async_copy(y_sp.at[i1_v[b]], v1_v[b], gsem[b])

        def drain_and_store(c, b):
            pltpu.make_async_copy(y_sp.at[i0_v[b]], v0_v[b], gsem[b]).wait()
            pltpu.make_async_copy(y_sp.at[i1_v[b]], v1_v[b], gsem[b]).wait()

            def mix_body(i, _):
                sl = pl.ds(i * L, L)
                v0 = v0_v[b][sl]
                v0_v[b][sl] = v0 + (v1_v[b][sl] - v0) * t_v[b][sl]
                return 0

            lax.fori_loop(0, CHUNK // L, mix_body, 0, unroll=8)
            base = core_base + c * piece + sid * CHUNK
            pltpu.sync_copy(v0_v[b], out_hbm.at[pl.ds(base, CHUNK)])

        stage_and_fire(0, 0)

        def pair_body(c2, _):
            s1 = 2 * c2 + 1

            @pl.when(s1 < n_chunks)
            def _():
                stage_and_fire(s1, 1)

            drain_and_store(s1 - 1, 0)

            @pl.when(s1 + 1 < n_chunks)
            def _():
                stage_and_fire(s1 + 1, 0)

            @pl.when(s1 < n_chunks)
            def _():
                drain_and_store(s1, 1)

            return 0

        lax.fori_loop(0, n_chunks // 2, pair_body, 0)

    return k


def kernel(t_query, t_knots, y):
    nq = t_query.shape[0]
    nk = t_knots.shape[0]
    return _make_kernel(nq, nk)(t_query, y)


# speedup vs baseline: 1.0692x; 1.0207x over previous
import functools

import jax
import jax.numpy as jnp
from jax import lax
from jax.experimental import pallas as pl
from jax.experimental.pallas import tpu as pltpu
from jax.experimental.pallas import tpu_sc as plsc

L = 16
NC = 2
NS = 16
CHUNK = 4096


def _make_kernel(nq, nk):
    piece = NS * CHUNK
    q_per_core = nq // NC
    n_chunks = q_per_core // piece
    assert n_chunks % 2 == 0 and n_chunks >= 4
    mesh = plsc.VectorSubcoreMesh(core_axis_name="c", subcore_axis_name="s")

    vm = lambda dt: pltpu.VMEM((CHUNK,), dt)

    @functools.partial(
        pl.kernel,
        out_type=jax.ShapeDtypeStruct((nq,), jnp.float32),
        mesh=mesh,
        scratch_types=[
            [vm(jnp.float32) for _ in range(2)],
            [vm(jnp.float32) for _ in range(2)],
            [vm(jnp.int32) for _ in range(2)],
            [vm(jnp.int32) for _ in range(2)],
            [vm(jnp.float32) for _ in range(2)],
            [vm(jnp.float32) for _ in range(2)],
            [pltpu.SemaphoreType.DMA for _ in range(2)],
            [pltpu.SemaphoreType.DMA for _ in range(2)],
            [pltpu.SemaphoreType.DMA for _ in range(2)],
            [pltpu.SemaphoreType.DMA for _ in range(2)],
            pltpu.VMEM_SHARED((nk,), jnp.float32),
            [pltpu.VMEM_SHARED((NS * CHUNK,), jnp.float32) for _ in range(2)],
        ],
    )
    def k(tq_hbm, y_hbm, out_hbm, t_v, f_v, i0_v, i1_v, v0_v, v1_v,
          gsem, tsem, csem, ssem, y_sp, t_sp):
        sid = lax.axis_index("s")
        cid = lax.axis_index("c")
        core_base = cid * q_per_core
        i_max = jnp.int32(nk - 2)

        def bulk_src(c):
            return tq_hbm.at[pl.ds(core_base + c * piece, piece)]

        def slice_src(slot):
            return t_sp[slot].at[pl.ds(sid * CHUNK, CHUNK)]

        def out_dst(c):
            return out_hbm.at[pl.ds(core_base + c * piece + sid * CHUNK, CHUNK)]

        def idx_loop(b):
            def body(i, _):
                sl = pl.ds(i * L, L)
                t = t_v[b][sl]
                i0 = jnp.minimum(t.astype(jnp.int32), i_max)
                i0_v[b][sl] = i0
                i1_v[b][sl] = i0 + 1
                f_v[b][sl] = t - i0.astype(jnp.float32)
                return 0

            lax.fori_loop(0, CHUNK // L, body, 0, unroll=8)

        def mix_loop(b):
            def body(i, _):
                sl = pl.ds(i * L, L)
                v0 = v0_v[b][sl]
                v0_v[b][sl] = v0 + (v1_v[b][sl] - v0) * f_v[b][sl]
                return 0

            lax.fori_loop(0, CHUNK // L, body, 0, unroll=8)

        def fire_gathers(b):
            pltpu.async_copy(y_sp.at[i0_v[b]], v0_v[b], gsem[b])
            pltpu.async_copy(y_sp.at[i1_v[b]], v1_v[b], gsem[b])

        def wait_gathers(b):
            pltpu.make_async_copy(y_sp.at[i0_v[b]], v0_v[b], gsem[b]).wait()
            pltpu.make_async_copy(y_sp.at[i1_v[b]], v1_v[b], gsem[b]).wait()

        def stage(s, b):
            pltpu.make_async_copy(slice_src(b), t_v[b], csem[b]).wait()

            @pl.when(jnp.logical_and(sid == 0, s + 1 < n_chunks))
            def _():
                pltpu.make_async_copy(bulk_src(s + 1), t_sp[1 - b], tsem[1 - b]).wait()

            plsc.subcore_barrier()

            @pl.when(jnp.logical_and(sid == 0, s + 2 < n_chunks))
            def _():
                pltpu.async_copy(bulk_src(s + 2), t_sp[b], tsem[b])

            idx_loop(b)

            @pl.when(s > 1)
            def _():
                pltpu.make_async_copy(v0_v[b], out_dst(s - 2), ssem[b]).wait()

            @pl.when(s + 1 < n_chunks)
            def _():
                pltpu.async_copy(slice_src(1 - b), t_v[1 - b], csem[1 - b])

            fire_gathers(b)

        def drain(c, b):
            wait_gathers(b)
            mix_loop(b)
            pltpu.async_copy(v0_v[b], out_dst(c), ssem[b])

        @pl.when(sid == 0)
        def _():
            pltpu.sync_copy(y_hbm, y_sp)
            pltpu.async_copy(bulk_src(0), t_sp[0], tsem[0])
            pltpu.async_copy(bulk_src(1), t_sp[1], tsem[1])

        @pl.when(sid == 0)
        def _():
            pltpu.make_async_copy(bulk_src(0), t_sp[0], tsem[0]).wait()
            pltpu.make_async_copy(bulk_src(1), t_sp[1], tsem[1]).wait()

        plsc.subcore_barrier()
        pltpu.sync_copy(slice_src(0), t_v[0])
        plsc.subcore_barrier()

        @pl.when(sid == 0)
        def _():
            pltpu.async_copy(bulk_src(2), t_sp[0], tsem[0])

        idx_loop(0)
        pltpu.async_copy(slice_src(1), t_v[1], csem[1])
        fire_gathers(0)

        def pair_body(c2, _):
            s1 = 2 * c2 + 1
            stage(s1, 1)
            drain(s1 - 1, 0)

            @pl.when(s1 + 1 < n_chunks)
            def _():
                stage(s1 + 1, 0)

            @pl.when(s1 < n_chunks - 1)
            def _():
                drain(s1, 1)

            return 0

        lax.fori_loop(0, n_chunks // 2, pair_body, 0)

        drain(n_chunks - 1, 1)
        pltpu.make_async_copy(v0_v[0], out_dst(n_chunks - 2), ssem[0]).wait()
        pltpu.make_async_copy(v0_v[1], out_dst(n_chunks - 1), ssem[1]).wait()

    return k


def kernel(t_query, t_knots, y):
    nq = t_query.shape[0]
    nk = t_knots.shape[0]
    return _make_kernel(nq, nk)(t_query, y)
